# baseline (speedup 1.0000x reference)
import functools

import jax
import jax.numpy as jnp
from jax import lax
from jax.experimental import pallas as pl
from jax.experimental.pallas import tpu as pltpu
from jax.experimental.pallas import tpu_sc as plsc

D = 32
L = 16
IW = 128
SUB = 4
CH = SUB * IW
N_WORKERS = 32
V = 1000000
BW = 512
NBUF = 2
N_FULL = V // BW
V_TAIL = N_FULL * BW
N_TAIL = V - V_TAIL


def _build_format_call():
    per_w = -(-N_FULL // N_WORKERS)
    per_w += (-per_w) % (NBUF // 2)
    n_iters = 2 * per_w
    mesh = plsc.VectorSubcoreMesh(core_axis_name="c", subcore_axis_name="s")
    num_cores = mesh.num_cores

    @functools.partial(
        pl.kernel,
        out_type=(jax.ShapeDtypeStruct((V * D,), jnp.float32),
                  jax.ShapeDtypeStruct((V * D,), jnp.float32)),
        mesh=mesh,
        compiler_params=pltpu.CompilerParams(needs_layout_passes=False),
        scratch_types=[
            [pltpu.VMEM((D // 8, 8, BW), jnp.float32) for _ in range(NBUF)],
            [pltpu.VMEM((BW * D,), jnp.float32) for _ in range(NBUF)],
            pltpu.VMEM((N_TAIL * D,), jnp.float32),
            pltpu.SemaphoreType.DMA,
            pltpu.SemaphoreType.DMA,
        ],
    )
    def fmt_call(uembT, iembT, utail, itail, uflat, iflat,
                 inb, outb, tailb, sem_in, sem_out):
        wid = lax.axis_index("s") * num_cores + lax.axis_index("c")
        base_blk = wid * per_w
        lane32 = lax.iota(jnp.int32, L) * D

        def vstart_of(s):
            blk = lax.min(base_blk + s // 2, N_FULL - 1)
            return pl.multiple_of(blk * BW, BW)

        def fire_in(s, p, src):
            vs = vstart_of(s)
            for t in range(D // 8):
                pltpu.async_copy(
                    src.at[t, :, pl.ds(vs, BW)], inb[p].at[t], sem_in)

        def wait_in(src):
            for t in range(D // 8):
                pltpu.make_async_copy(
                    src.at[t, :, pl.ds(0, BW)], inb[0].at[t], sem_in).wait()

        def fire_out(s, p, dst):
            pltpu.async_copy(
                outb[p], dst.at[pl.ds(vstart_of(s) * D, BW * D)], sem_out)

        def wait_out(dst):
            pltpu.make_async_copy(
                outb[0], dst.at[pl.ds(0, BW * D)], sem_out).wait()

        srcs = (uembT, iembT)
        dsts = (uflat, iflat)

        @pl.when(wid == 0)
        def _():
            for tail, dst in ((utail, uflat), (itail, iflat)):
                pltpu.sync_copy(tail, tailb)
                pltpu.sync_copy(tailb, dst.at[pl.ds(V_TAIL * D, N_TAIL * D)])

        for p in range(NBUF):
            fire_in(p, p, srcs[p % 2])

        def iter_body(s, p):
            src = srcs[p % 2]
            dst = dsts[p % 2]
            wait_in(src)

            @pl.when(s >= NBUF)
            def _():
                wait_out(dst)

            ib = inb[p]
            ob = outb[p]

            def d_body(d):
                for vv0 in range(0, BW, L):
                    vec = ib[d // 8, d % 8, pl.ds(vv0, L)]
                    plsc.store_scatter(ob, [lane32 + (vv0 * D + d)], vec)

            plsc.parallel_loop(0, D, unroll=4)(d_body)

            @pl.when(s + NBUF < n_iters)
            def _():
                fire_in(s + NBUF, p, src)

            fire_out(s, p, dst)

        def ring_body(t):
            for p in range(NBUF):
                iter_body(t * NBUF + p, p)

        pl.loop(0, n_iters // NBUF)(ring_body)
        for p in range(NBUF):
            wait_out(dsts[p % 2])

    return fmt_call


def _build_gather_call(B):
    n_per_w = B // N_WORKERS
    n_iters = n_per_w // CH
    n_rows_w = n_per_w // IW
    mesh = plsc.VectorSubcoreMesh(core_axis_name="c", subcore_axis_name="s")
    num_cores = mesh.num_cores

    @functools.partial(
        pl.kernel,
        out_type=jax.ShapeDtypeStruct((B,), jnp.float32),
        mesh=mesh,
        compiler_params=pltpu.CompilerParams(
            needs_layout_passes=False, use_tc_tiling_on_sc=False),
        scratch_types=[
            [pltpu.VMEM((SUB, IW), jnp.int32) for _ in range(2)],
            [pltpu.VMEM((SUB, IW), jnp.int32) for _ in range(2)],
            [pltpu.VMEM((CH, D), jnp.float32) for _ in range(2)],
            [pltpu.VMEM((CH, D), jnp.float32) for _ in range(2)],
            [pltpu.VMEM((CH,), jnp.float32) for _ in range(2)],
            pltpu.SemaphoreType.DMA,
            pltpu.SemaphoreType.DMA,
            pltpu.SemaphoreType.DMA,
        ],
    )
    def sc_call(uid_hbm, iid_hbm, uemb_hbm, iemb_hbm, out_hbm,
                uidx, iidx, urows, irows, outb, sem_idx, sem_rows, sem_out):
        wid = lax.axis_index("s") * num_cores + lax.axis_index("c")
        wrow = wid * n_rows_w
        base = wid * n_per_w
        lane = lax.iota(jnp.int32, L)
        lane0 = lane == 0

        def fire_idx(s, p):
            goff = wrow + s * SUB
            pltpu.async_copy(uid_hbm.at[pl.ds(goff, SUB)], uidx[p], sem_idx)
            pltpu.async_copy(iid_hbm.at[pl.ds(goff, SUB)], iidx[p], sem_idx)

        def wait_idx():
            pltpu.make_async_copy(
                uid_hbm.at[pl.ds(0, SUB)], uidx[0], sem_idx).wait()
            pltpu.make_async_copy(
                iid_hbm.at[pl.ds(0, SUB)], iidx[0], sem_idx).wait()

        def fire_rows(p):
            for r in range(SUB):
                pltpu.async_copy(
                    uemb_hbm.at[uidx[p].at[r]],
                    urows[p].at[pl.ds(r * IW, IW)], sem_rows)
                pltpu.async_copy(
                    iemb_hbm.at[iidx[p].at[r]],
                    irows[p].at[pl.ds(r * IW, IW)], sem_rows)

        def wait_rows():
            pltpu.make_async_copy(
                uemb_hbm.at[pl.ds(0, CH)], urows[0], sem_rows).wait()
            pltpu.make_async_copy(
                iemb_hbm.at[pl.ds(0, CH)], irows[0], sem_rows).wait()

        def wait_out():
            pltpu.make_async_copy(
                outb[0], out_hbm.at[pl.ds(0, CH)], sem_out).wait()

        fire_idx(0, 0)
        fire_idx(1, 1)
        wait_idx()
        fire_rows(0)

        def iter_body(s, par):
            nxt = 1 - par
            wait_rows()

            @pl.when(s + 1 < n_iters)
            def _():
                wait_idx()
                fire_rows(nxt)

            @pl.when(s + 2 < n_iters)
            def _():
                fire_idx(s + 2, par)

            @pl.when(s >= 2)
            def _():
                wait_out()

            ur = urows[par]
            ir = irows[par]
            ob = outb[par]

            def out_body(b):
                u0 = ur[b, pl.ds(0, L)]
                u1 = ur[b, pl.ds(L, L)]
                v0 = ir[b, pl.ds(0, L)]
                v1 = ir[b, pl.ds(L, L)]
                s_ = jnp.sum(u0 * v0 + u1 * v1)
                plsc.store_scatter(
                    ob,
                    [jnp.full((L,), b, jnp.int32)],
                    jnp.full((L,), s_, jnp.float32),
                    mask=lane0,
                )

            plsc.parallel_loop(0, CH, unroll=8)(out_body)
            pltpu.async_copy(ob, out_hbm.at[pl.ds(base + s * CH, CH)], sem_out)

        def pair_body(t):
            iter_body(2 * t, 0)
            iter_body(2 * t + 1, 1)

        pl.loop(0, n_iters // 2)(pair_body)

        wait_out()
        wait_out()

    return sc_call


@jax.jit
def _listmf(uid, iid, user_emb, item_emb):
    B = uid.size
    fmt_call = _build_format_call()
    utail = user_emb[V_TAIL:].reshape(-1)
    itail = item_emb[V_TAIL:].reshape(-1)
    uflat, iflat = fmt_call(user_emb.T.reshape(D // 8, 8, V),
                            item_emb.T.reshape(D // 8, 8, V), utail, itail)
    sc_call = _build_gather_call(B)
    return sc_call(uid.reshape(B // IW, IW), iid.reshape(B // IW, IW),
                   uflat.reshape(V, D), iflat.reshape(V, D))


def kernel(userID, itemID, rels, mode, user_emb, item_emb):
    shape = userID.shape
    uid = jnp.asarray(userID, jnp.int32).reshape(-1)
    iid = jnp.asarray(itemID, jnp.int32).reshape(-1)
    out = _listmf(uid, iid, user_emb, item_emb)
    return out.reshape(shape)


# speedup vs baseline: 3.0287x; 3.0287x over previous
import functools

import jax
import jax.numpy as jnp
from jax import lax
from jax.experimental import pallas as pl
from jax.experimental.pallas import tpu as pltpu
from jax.experimental.pallas import tpu_sc as plsc

D = 32
L = 16
IW = 128
SUB = 4
CH = SUB * IW
N_WORKERS = 32
V = 1000000
BW = 512
NBUF = 2
N_FULL = V // BW
V_TAIL = N_FULL * BW
N_TAIL = V - V_TAIL


def _build_format_call():
    per_w = -(-N_FULL // N_WORKERS)
    per_w += (-per_w) % (NBUF // 2)
    n_iters = 2 * per_w
    mesh = plsc.VectorSubcoreMesh(core_axis_name="c", subcore_axis_name="s")
    num_cores = mesh.num_cores

    @functools.partial(
        pl.kernel,
        out_type=(jax.ShapeDtypeStruct((V * D,), jnp.float32),
                  jax.ShapeDtypeStruct((V * D,), jnp.float32)),
        mesh=mesh,
        compiler_params=pltpu.CompilerParams(needs_layout_passes=False),
        scratch_types=[
            [pltpu.VMEM((D // 8, 8, BW), jnp.float32) for _ in range(NBUF)],
            [pltpu.VMEM((BW * D,), jnp.float32) for _ in range(NBUF)],
            pltpu.VMEM((BW * (D + 1) + L,), jnp.float32),
            pltpu.VMEM((N_TAIL * D,), jnp.float32),
            pltpu.SemaphoreType.DMA,
            pltpu.SemaphoreType.DMA,
        ],
    )
    def fmt_call(uembT, iembT, utail, itail, uflat, iflat,
                 inb, outb, stg, tailb, sem_in, sem_out):
        wid = lax.axis_index("s") * num_cores + lax.axis_index("c")
        base_blk = wid * per_w
        lane = lax.iota(jnp.int32, L)
        lane33 = lane * (D + 1)

        def vstart_of(s):
            blk = lax.min(base_blk + s // 2, N_FULL - 1)
            return pl.multiple_of(blk * BW, BW)

        def fire_in(s, p, src):
            vs = vstart_of(s)
            for t in range(D // 8):
                pltpu.async_copy(
                    src.at[t, :, pl.ds(vs, BW)], inb[p].at[t], sem_in)

        def wait_in(src):
            for t in range(D // 8):
                pltpu.make_async_copy(
                    src.at[t, :, pl.ds(0, BW)], inb[0].at[t], sem_in).wait()

        def fire_out(s, p, dst):
            pltpu.async_copy(
                outb[p], dst.at[pl.ds(vstart_of(s) * D, BW * D)], sem_out)

        def wait_out(dst):
            pltpu.make_async_copy(
                outb[0], dst.at[pl.ds(0, BW * D)], sem_out).wait()

        srcs = (uembT, iembT)
        dsts = (uflat, iflat)

        @pl.when(wid == 0)
        def _():
            for tail, dst in ((utail, uflat), (itail, iflat)):
                pltpu.sync_copy(tail, tailb)
                pltpu.sync_copy(tailb, dst.at[pl.ds(V_TAIL * D, N_TAIL * D)])

        for p in range(NBUF):
            fire_in(p, p, srcs[p % 2])

        def iter_body(s, p):
            src = srcs[p % 2]
            dst = dsts[p % 2]
            wait_in(src)

            @pl.when(s >= NBUF)
            def _():
                wait_out(dst)

            ib = inb[p]
            ob = outb[p]

            def d_body(d):
                for vv0 in range(0, BW, L):
                    vec = ib[d // 8, d % 8, pl.ds(vv0, L)]
                    plsc.store_scatter(
                        stg, [lane33 + (vv0 * (D + 1) + d)], vec)

            plsc.parallel_loop(0, D, unroll=4)(d_body)

            def v_body(vv):
                i0 = lane + vv * (D + 1)
                ob[pl.ds(vv * D, L)] = plsc.load_gather(stg, [i0])
                ob[pl.ds(vv * D + L, L)] = plsc.load_gather(stg, [i0 + L])

            plsc.parallel_loop(0, BW, unroll=8)(v_body)

            @pl.when(s + NBUF < n_iters)
            def _():
                fire_in(s + NBUF, p, src)

            fire_out(s, p, dst)

        def ring_body(t):
            for p in range(NBUF):
                iter_body(t * NBUF + p, p)

        pl.loop(0, n_iters // NBUF)(ring_body)
        for p in range(NBUF):
            wait_out(dsts[p % 2])

    return fmt_call


def _build_gather_call(B):
    n_per_w = B // N_WORKERS
    n_iters = n_per_w // CH
    n_rows_w = n_per_w // IW
    mesh = plsc.VectorSubcoreMesh(core_axis_name="c", subcore_axis_name="s")
    num_cores = mesh.num_cores

    @functools.partial(
        pl.kernel,
        out_type=jax.ShapeDtypeStruct((B,), jnp.float32),
        mesh=mesh,
        compiler_params=pltpu.CompilerParams(
            needs_layout_passes=False, use_tc_tiling_on_sc=False),
        scratch_types=[
            [pltpu.VMEM((SUB, IW), jnp.int32) for _ in range(2)],
            [pltpu.VMEM((SUB, IW), jnp.int32) for _ in range(2)],
            [pltpu.VMEM((CH, D), jnp.float32) for _ in range(2)],
            [pltpu.VMEM((CH, D), jnp.float32) for _ in range(2)],
            [pltpu.VMEM((CH,), jnp.float32) for _ in range(2)],
            pltpu.SemaphoreType.DMA,
            pltpu.SemaphoreType.DMA,
            pltpu.SemaphoreType.DMA,
        ],
    )
    def sc_call(uid_hbm, iid_hbm, uemb_hbm, iemb_hbm, out_hbm,
                uidx, iidx, urows, irows, outb, sem_idx, sem_rows, sem_out):
        wid = lax.axis_index("s") * num_cores + lax.axis_index("c")
        wrow = wid * n_rows_w
        base = wid * n_per_w
        lane = lax.iota(jnp.int32, L)
        lane0 = lane == 0

        def fire_idx(s, p):
            goff = wrow + s * SUB
            pltpu.async_copy(uid_hbm.at[pl.ds(goff, SUB)], uidx[p], sem_idx)
            pltpu.async_copy(iid_hbm.at[pl.ds(goff, SUB)], iidx[p], sem_idx)

        def wait_idx():
            pltpu.make_async_copy(
                uid_hbm.at[pl.ds(0, SUB)], uidx[0], sem_idx).wait()
            pltpu.make_async_copy(
                iid_hbm.at[pl.ds(0, SUB)], iidx[0], sem_idx).wait()

        def fire_rows(p):
            for r in range(SUB):
                pltpu.async_copy(
                    uemb_hbm.at[uidx[p].at[r]],
                    urows[p].at[pl.ds(r * IW, IW)], sem_rows)
                pltpu.async_copy(
                    iemb_hbm.at[iidx[p].at[r]],
                    irows[p].at[pl.ds(r * IW, IW)], sem_rows)

        def wait_rows():
            pltpu.make_async_copy(
                uemb_hbm.at[pl.ds(0, CH)], urows[0], sem_rows).wait()
            pltpu.make_async_copy(
                iemb_hbm.at[pl.ds(0, CH)], irows[0], sem_rows).wait()

        def wait_out():
            pltpu.make_async_copy(
                outb[0], out_hbm.at[pl.ds(0, CH)], sem_out).wait()

        fire_idx(0, 0)
        fire_idx(1, 1)
        wait_idx()
        fire_rows(0)

        def iter_body(s, par):
            nxt = 1 - par
            wait_rows()

            @pl.when(s + 1 < n_iters)
            def _():
                wait_idx()
                fire_rows(nxt)

            @pl.when(s + 2 < n_iters)
            def _():
                fire_idx(s + 2, par)

            @pl.when(s >= 2)
            def _():
                wait_out()

            ur = urows[par]
            ir = irows[par]
            ob = outb[par]

            def out_body(b):
                u0 = ur[b, pl.ds(0, L)]
                u1 = ur[b, pl.ds(L, L)]
                v0 = ir[b, pl.ds(0, L)]
                v1 = ir[b, pl.ds(L, L)]
                s_ = jnp.sum(u0 * v0 + u1 * v1)
                plsc.store_scatter(
                    ob,
                    [jnp.full((L,), b, jnp.int32)],
                    jnp.full((L,), s_, jnp.float32),
                    mask=lane0,
                )

            plsc.parallel_loop(0, CH, unroll=8)(out_body)
            pltpu.async_copy(ob, out_hbm.at[pl.ds(base + s * CH, CH)], sem_out)

        def pair_body(t):
            iter_body(2 * t, 0)
            iter_body(2 * t + 1, 1)

        pl.loop(0, n_iters // 2)(pair_body)

        wait_out()
        wait_out()

    return sc_call


@jax.jit
def _listmf(uid, iid, user_emb, item_emb):
    B = uid.size
    fmt_call = _build_format_call()
    utail = user_emb[V_TAIL:].reshape(-1)
    itail = item_emb[V_TAIL:].reshape(-1)
    uflat, iflat = fmt_call(user_emb.T.reshape(D // 8, 8, V),
                            item_emb.T.reshape(D // 8, 8, V), utail, itail)
    sc_call = _build_gather_call(B)
    return sc_call(uid.reshape(B // IW, IW), iid.reshape(B // IW, IW),
                   uflat.reshape(V, D), iflat.reshape(V, D))


def kernel(userID, itemID, rels, mode, user_emb, item_emb):
    shape = userID.shape
    uid = jnp.asarray(userID, jnp.int32).reshape(-1)
    iid = jnp.asarray(itemID, jnp.int32).reshape(-1)
    out = _listmf(uid, iid, user_emb, item_emb)
    return out.reshape(shape)
